# baseline (speedup 1.0000x reference)
import jax
import jax.numpy as jnp
from jax import lax
from jax.experimental import pallas as pl
from jax.experimental.pallas import tpu as pltpu
from jax.experimental.pallas import tpu_sc as plsc

_BIT = 512
_SPIKE = 7.0

_ROWS = 384
_BLOCK = 24
_STEPS = _ROWS // _BLOCK
_M = 224
_NCHUNK = _M // 16


def _quant16(v):
    kf = jnp.clip(v / _SPIKE + 0.5, 0.0, float(_BIT - 1))
    k0 = kf.astype(jnp.int32)
    km = jnp.maximum(k0 - 1, 0)
    kp = jnp.minimum(k0 + 1, _BIT - 1)

    def dist(k):
        return jnp.abs(v - k.astype(jnp.float32) * _SPIKE)

    dm = dist(km)
    d0 = dist(k0)
    dp = dist(kp)
    best = km
    bd = dm
    t0 = d0 < bd
    best = jnp.where(t0, k0, best)
    bd = jnp.where(t0, d0, bd)
    tp = dp < bd
    best = jnp.where(tp, kp, best)
    return best.astype(jnp.float32) * _SPIKE


def _scalar_reduce(vec, redbuf, combine):
    del redbuf
    acc = vec[0]
    for i in range(1, 16):
        acc = combine(acc, vec[i])
    return acc


def _sc_quant_body(y_hbm, vals_hbm, ybuf, valsbuf, redbuf, redbuf2):
    wid = lax.axis_index("s") * 2 + lax.axis_index("c")

    @pl.when(wid == 0)
    def _():
        pltpu.sync_copy(y_hbm, ybuf)
        chunks = [ybuf[pl.ds(16 * i, 16)] for i in range(_NCHUNK)]
        mn = chunks[0]
        mx = chunks[0]
        for c in chunks[1:]:
            mn = jnp.minimum(mn, c)
            mx = jnp.maximum(mx, c)
        ymin = _scalar_reduce(mn, redbuf, jnp.minimum)
        ymax = _scalar_reduce(mx, redbuf2, jnp.maximum)
        rng = ymax - ymin
        for i in range(_NCHUNK):
            v = chunks[i] / rng * _SPIKE * float(_BIT)
            valsbuf[pl.ds(16 * i, 16)] = _quant16(v)
        pltpu.sync_copy(valsbuf, vals_hbm)


def _sc_quant(y):
    fn = pl.kernel(
        _sc_quant_body,
        out_type=jax.ShapeDtypeStruct((_M,), jnp.float32),
        mesh=plsc.VectorSubcoreMesh(core_axis_name="c", subcore_axis_name="s"),
        scratch_types=[
            pltpu.VMEM((_M,), jnp.float32),
            pltpu.VMEM((_M,), jnp.float32),
            pltpu.VMEM((16,), jnp.float32),
            pltpu.VMEM((16,), jnp.float32),
        ],
    )
    return fn(y)


def _fused_body(x_ref, vals_in_ref, o_ref, mm_ref, vals_ref):
    p = pl.program_id(0)
    j = pl.program_id(1)

    @pl.when(p == 0)
    def _reduce():
        bmin = jnp.min(x_ref[...])
        bmax = jnp.max(x_ref[...])

        @pl.when(j == 0)
        def _init():
            mm_ref[0] = bmin
            mm_ref[1] = bmax

        @pl.when(j > 0)
        def _acc():
            mm_ref[0] = jnp.minimum(mm_ref[0], bmin)
            mm_ref[1] = jnp.maximum(mm_ref[1], bmax)

    @pl.when((p == 1) & (j == 0))
    def _mask():
        n = (mm_ref[1] - mm_ref[0]).astype(jnp.int32) + 1
        row = jax.lax.broadcasted_iota(jnp.int32, (_M, 1), 0)
        vals_ref[...] = jnp.where(row < n, vals_in_ref[...], 0.0)

    @pl.when(p == 1)
    def _emit():
        o_ref[...] = jnp.broadcast_to(vals_ref[...][None], o_ref.shape)


def kernel(x, y):
    vals = _sc_quant(y)
    out3 = pl.pallas_call(
        _fused_body,
        grid=(2, _STEPS),
        in_specs=[
            pl.BlockSpec(
                (_BLOCK, 224, 224),
                lambda p, j: (j * (1 - p) + (_STEPS - 1) * p, 0, 0),
            ),
            pl.BlockSpec((_M, 1), lambda p, j: (0, 0)),
        ],
        out_specs=pl.BlockSpec((_BLOCK, 224, 224), lambda p, j: (j * p, 0, 0)),
        out_shape=jax.ShapeDtypeStruct((_ROWS, 224, 224), jnp.float32),
        scratch_shapes=[
            pltpu.SMEM((2,), jnp.float32),
            pltpu.VMEM((_M, 1), jnp.float32),
        ],
    )(x.reshape(_ROWS, 224, 224), vals.reshape(_M, 1))
    return out3.reshape(x.shape)


# speedup vs baseline: 1.0420x; 1.0420x over previous
import jax
import jax.numpy as jnp
from jax import lax
from jax.experimental import pallas as pl
from jax.experimental.pallas import tpu as pltpu
from jax.experimental.pallas import tpu_sc as plsc

_BIT = 512
_SPIKE = 7.0

_ROWS = 384
_BLOCK = 24
_STEPS = _ROWS // _BLOCK
_M = 224
_NCHUNK = _M // 16


def _quant16(v):
    kf = jnp.clip(v / _SPIKE + 0.5, 0.0, float(_BIT - 1))
    k0 = kf.astype(jnp.int32)
    km = jnp.maximum(k0 - 1, 0)
    kp = jnp.minimum(k0 + 1, _BIT - 1)

    def dist(k):
        return jnp.abs(v - k.astype(jnp.float32) * _SPIKE)

    dm = dist(km)
    d0 = dist(k0)
    dp = dist(kp)
    best = km
    bd = dm
    t0 = d0 < bd
    best = jnp.where(t0, k0, best)
    bd = jnp.where(t0, d0, bd)
    tp = dp < bd
    best = jnp.where(tp, kp, best)
    return best.astype(jnp.float32) * _SPIKE


def _scalar_reduce(vec, combine):
    acc = vec[0]
    for i in range(1, 16):
        acc = combine(acc, vec[i])
    return acc


def _sc_quant_body(y_hbm, vals_hbm, ybuf, valsbuf):
    wid = lax.axis_index("s") * 2 + lax.axis_index("c")

    @pl.when(wid == 0)
    def _():
        pltpu.sync_copy(y_hbm, ybuf)
        chunks = [ybuf[pl.ds(16 * i, 16)] for i in range(_NCHUNK)]
        mn = chunks[0]
        mx = chunks[0]
        for c in chunks[1:]:
            mn = jnp.minimum(mn, c)
            mx = jnp.maximum(mx, c)
        ymin = _scalar_reduce(mn, jnp.minimum)
        ymax = _scalar_reduce(mx, jnp.maximum)
        rng = ymax - ymin
        for i in range(_NCHUNK):
            v = chunks[i] / rng * _SPIKE * float(_BIT)
            valsbuf[pl.ds(16 * i, 16)] = _quant16(v)
        pltpu.sync_copy(valsbuf, vals_hbm)


def _sc_quant(y):
    fn = pl.kernel(
        _sc_quant_body,
        out_type=jax.ShapeDtypeStruct((_M,), jnp.float32),
        mesh=plsc.VectorSubcoreMesh(core_axis_name="c", subcore_axis_name="s"),
        scratch_types=[
            pltpu.VMEM((_M,), jnp.float32),
            pltpu.VMEM((_M,), jnp.float32),
        ],
    )
    return fn(y)


def _minmax_body(x_ref, mm_ref):
    j = pl.program_id(0)
    bmin = jnp.min(x_ref[...])
    bmax = jnp.max(x_ref[...])

    @pl.when(j == 0)
    def _init():
        mm_ref[0] = bmin
        mm_ref[1] = bmax

    @pl.when(j > 0)
    def _acc():
        mm_ref[0] = jnp.minimum(mm_ref[0], bmin)
        mm_ref[1] = jnp.maximum(mm_ref[1], bmax)


def _bcast_body(vals_in_ref, mm_ref, o_ref, vals_ref):
    j = pl.program_id(0)

    @pl.when(j == 0)
    def _mask():
        n = (mm_ref[1] - mm_ref[0]).astype(jnp.int32) + 1
        row = jax.lax.broadcasted_iota(jnp.int32, (_M, 1), 0)
        vals_ref[...] = jnp.where(row < n, vals_in_ref[...], 0.0)

    o_ref[...] = jnp.broadcast_to(vals_ref[...][None], o_ref.shape)


def kernel(x, y):
    vals = _sc_quant(y)
    x3 = x.reshape(_ROWS, 224, 224)
    mm = pl.pallas_call(
        _minmax_body,
        grid=(_STEPS,),
        in_specs=[pl.BlockSpec((_BLOCK, 224, 224), lambda j: (j, 0, 0))],
        out_specs=pl.BlockSpec(memory_space=pltpu.SMEM),
        out_shape=jax.ShapeDtypeStruct((2,), jnp.float32),
    )(x3)
    out3 = pl.pallas_call(
        _bcast_body,
        grid=(_STEPS,),
        in_specs=[
            pl.BlockSpec((_M, 1), lambda j: (0, 0)),
            pl.BlockSpec(memory_space=pltpu.SMEM),
        ],
        out_specs=pl.BlockSpec((_BLOCK, 224, 224), lambda j: (j, 0, 0)),
        out_shape=jax.ShapeDtypeStruct((_ROWS, 224, 224), jnp.float32),
        scratch_shapes=[pltpu.VMEM((_M, 1), jnp.float32)],
    )(vals.reshape(_M, 1), mm)
    return out3.reshape(x.shape)


# speedup vs baseline: 1.3901x; 1.3340x over previous
import jax
import jax.numpy as jnp
from jax.experimental import pallas as pl
from jax.experimental.pallas import tpu as pltpu

_BIT = 512
_SPIKE = 7.0

_ROWS = 384
_BLOCK = 24
_STEPS = _ROWS // _BLOCK
_NBUF = 4
_M = 224


def _quant(v):
    kf = jnp.clip(v / _SPIKE + 0.5, 0.0, float(_BIT - 1))
    k0 = kf.astype(jnp.int32)
    km = jnp.maximum(k0 - 1, 0)
    kp = jnp.minimum(k0 + 1, _BIT - 1)

    def dist(k):
        return jnp.abs(v - k.astype(jnp.float32) * _SPIKE)

    dm = dist(km)
    d0 = dist(k0)
    dp = dist(kp)
    best = km
    bd = dm
    t0 = d0 < bd
    best = jnp.where(t0, k0, best)
    bd = jnp.where(t0, d0, bd)
    tp = dp < bd
    best = jnp.where(tp, kp, best)
    return best.astype(jnp.float32) * _SPIKE


def _body(x_hbm, y_ref, o_hbm, xbuf, pat, rsems, wsem):
    def read(j, slot):
        pltpu.make_async_copy(
            x_hbm.at[pl.ds(j * _BLOCK, _BLOCK)],
            xbuf.at[slot],
            rsems.at[slot],
        ).start()

    for b in range(_NBUF):
        read(b, b)

    mn = None
    mx = None
    for j in range(_STEPS):
        slot = j % _NBUF
        pltpu.make_async_copy(
            x_hbm.at[pl.ds(j * _BLOCK, _BLOCK)],
            xbuf.at[slot],
            rsems.at[slot],
        ).wait()
        blk = xbuf[slot]
        bmn = jnp.min(blk)
        bmx = jnp.max(blk)
        mn = bmn if mn is None else jnp.minimum(mn, bmn)
        mx = bmx if mx is None else jnp.maximum(mx, bmx)
        if j + _NBUF < _STEPS:
            read(j + _NBUF, slot)

    y = y_ref[...]
    ymax = jnp.max(y)
    ymin = jnp.min(y)
    v = y / (ymax - ymin) * _SPIKE * float(_BIT)
    vals = _quant(v)
    n = (mx - mn).astype(jnp.int32) + 1
    row = jax.lax.broadcasted_iota(jnp.int32, (_M, 1), 0)
    vals = jnp.where(row < n, vals, 0.0)
    pat[...] = jnp.broadcast_to(vals[None], pat.shape)

    for j in range(_STEPS):
        pltpu.make_async_copy(
            pat, o_hbm.at[pl.ds(j * _BLOCK, _BLOCK)], wsem
        ).start()
    for j in range(_STEPS):
        pltpu.make_async_copy(
            pat, o_hbm.at[pl.ds(j * _BLOCK, _BLOCK)], wsem
        ).wait()


def kernel(x, y):
    out3 = pl.pallas_call(
        _body,
        in_specs=[
            pl.BlockSpec(memory_space=pl.ANY),
            pl.BlockSpec(memory_space=pltpu.VMEM),
        ],
        out_specs=pl.BlockSpec(memory_space=pl.ANY),
        out_shape=jax.ShapeDtypeStruct((_ROWS, 224, 224), jnp.float32),
        scratch_shapes=[
            pltpu.VMEM((_NBUF, _BLOCK, 224, 224), jnp.float32),
            pltpu.VMEM((_BLOCK, 224, 224), jnp.float32),
            pltpu.SemaphoreType.DMA((_NBUF,)),
            pltpu.SemaphoreType.DMA,
        ],
    )(x.reshape(_ROWS, 224, 224), y.reshape(_M, 1))
    return out3.reshape(x.shape)
